# Initial kernel scaffold; baseline (speedup 1.0000x reference)
import jax
import jax.numpy as jnp
from jax.experimental import pallas as pl


def kernel(x, w_fused, b_fused):
    raise NotImplementedError("write your pallas kernel here")


# speedup vs baseline: 1.0286x; 1.0286x over previous
import jax
import jax.numpy as jnp
from jax.experimental import pallas as pl
from jax.experimental.pallas import tpu as pltpu

L = 50
OUT = 10
TBG = 2048


def _gate_kernel(x_ref, w_ref, b_ref, s_ref, o_ref):
    y = jnp.dot(x_ref[...], w_ref[...], preferred_element_type=jnp.float32)
    y = jax.nn.sigmoid(y + b_ref[...])
    e = jnp.exp(y)
    denom = jnp.dot(e, s_ref[...], preferred_element_type=jnp.float32)
    o_ref[...] = (e * pl.reciprocal(denom, approx=True)).astype(o_ref.dtype)


def kernel(x, w_fused, b_fused):
    B = x.shape[0]
    assert x.shape[1] == 1 and x.shape[2] == L
    x2 = x.reshape(B, L).astype(jnp.float32)

    if B % 8 == 0:
        G = 8
    elif B % 2 == 0:
        G = 2
    else:
        G = 1
    M = B // G
    xg = x2.reshape(M, G * L)

    eye = jnp.eye(G, dtype=jnp.float32)
    w_big = jnp.kron(eye, w_fused.astype(jnp.float32))
    b_big = jnp.tile(b_fused.astype(jnp.float32), (1, G))
    s_big = jnp.kron(eye, jnp.ones((OUT, OUT), jnp.float32))

    tb = min(M, TBG)
    grid = (pl.cdiv(M, tb),)

    out = pl.pallas_call(
        _gate_kernel,
        out_shape=jax.ShapeDtypeStruct((M, G * OUT), jnp.float32),
        grid=grid,
        in_specs=[
            pl.BlockSpec((tb, G * L), lambda i: (i, 0)),
            pl.BlockSpec((G * L, G * OUT), lambda i: (0, 0)),
            pl.BlockSpec((1, G * OUT), lambda i: (0, 0)),
            pl.BlockSpec((G * OUT, G * OUT), lambda i: (0, 0)),
        ],
        out_specs=pl.BlockSpec((tb, G * OUT), lambda i: (i, 0)),
        compiler_params=pltpu.CompilerParams(
            dimension_semantics=("parallel",)),
    )(xg, w_big, b_big, s_big)

    return out.reshape(B, 1, OUT)
